# baseline (speedup 1.0000x reference)
import functools

import jax
import jax.numpy as jnp
from jax import lax
from jax.experimental import pallas as pl
from jax.experimental.pallas import tpu as pltpu
from jax.experimental.pallas import tpu_sc as plsc

_N = 10000
_D = 128
_NP = 10112
_NW = 32
_CH = 128
_NCH = 80
_HCH = 80
_EP = _NW * _NCH * _CH
_BN_EPS = 1e-5
_BLK = 1000
_NB = _N // _BLK


def _sc_agg_body(h_hbm, srcs_hbm, dsts_hbm, zeros_hbm, out_hbm,
                 src_v, dst_v, rows0_v, acc_sh, sem0):
    cid = lax.axis_index("c")
    sid = lax.axis_index("s")
    wid = sid * 2 + cid
    rz = _NP // 16

    pltpu.sync_copy(zeros_hbm.at[pl.ds(sid * rz, rz)],
                    acc_sh.at[pl.ds(sid * rz, rz)])
    plsc.subcore_barrier()

    def phase(p, carry):
        pltpu.sync_copy(srcs_hbm.at[wid, p], src_v)
        pltpu.sync_copy(dsts_hbm.at[wid, p], dst_v)

        def chunk(c, carry2):
            pltpu.async_copy(h_hbm.at[src_v.at[c]], rows0_v, sem0).wait()
            pltpu.sync_copy(rows0_v, acc_sh.at[dst_v.at[c]], add=True)
            return carry2

        lax.fori_loop(0, _HCH, chunk, 0)
        return carry

    lax.fori_loop(0, _NCH // _HCH, phase, 0)
    plsc.subcore_barrier()
    pltpu.sync_copy(acc_sh.at[pl.ds(sid * rz, rz)],
                    out_hbm.at[cid, pl.ds(sid * rz, rz)])


@jax.jit
def _sc_agg(h, srcs, dsts, zeros):
    mesh = plsc.VectorSubcoreMesh(core_axis_name="c", subcore_axis_name="s")
    return pl.kernel(
        _sc_agg_body,
        out_type=jax.ShapeDtypeStruct((2, _NP, _D), jnp.float32),
        mesh=mesh,
        scratch_types=[
            pltpu.VMEM((_HCH, _CH), jnp.int32),
            pltpu.VMEM((_HCH, _CH), jnp.int32),
            pltpu.VMEM((_CH, _D), jnp.float32),
            pltpu.VMEM_SHARED((_NP, _D), jnp.float32),
            pltpu.SemaphoreType.DMA,
        ],
    )(h, srcs, dsts, zeros)


def _mlp_body(eps_ref, x_ref, p0_ref, p1_ref, wa_ref, ba_ref, wb_ref, bb_ref,
              hpre_ref, stats_ref, acc_ref):
    h = x_ref[...] * (1.0 + eps_ref[0]) + p0_ref[...] + p1_ref[...]
    h = jnp.maximum(
        jnp.dot(h, wa_ref[...], preferred_element_type=jnp.float32)
        + ba_ref[...], 0.0)
    h = jnp.maximum(
        jnp.dot(h, wb_ref[...], preferred_element_type=jnp.float32)
        + bb_ref[...], 0.0)
    hpre_ref[...] = h
    i = pl.program_id(0)

    @pl.when(i == 0)
    def _():
        acc_ref[...] = jnp.zeros_like(acc_ref)

    acc_ref[0:1, :] += jnp.sum(h, axis=0, keepdims=True)
    acc_ref[1:2, :] += jnp.sum(h * h, axis=0, keepdims=True)

    @pl.when(i == pl.num_programs(0) - 1)
    def _():
        stats_ref[...] = acc_ref[...]


@jax.jit
def _mlp(eps, x, p0, p1, wa, ba, wb, bb):
    return pl.pallas_call(
        _mlp_body,
        grid=(_NB,),
        in_specs=[
            pl.BlockSpec(memory_space=pltpu.SMEM),
            pl.BlockSpec((_BLK, _D), lambda i: (i, 0)),
            pl.BlockSpec((_BLK, _D), lambda i: (i, 0)),
            pl.BlockSpec((_BLK, _D), lambda i: (i, 0)),
            pl.BlockSpec((_D, _D), lambda i: (0, 0)),
            pl.BlockSpec((1, _D), lambda i: (0, 0)),
            pl.BlockSpec((_D, _D), lambda i: (0, 0)),
            pl.BlockSpec((1, _D), lambda i: (0, 0)),
        ],
        out_specs=[
            pl.BlockSpec((_BLK, _D), lambda i: (i, 0)),
            pl.BlockSpec((8, _D), lambda i: (0, 0)),
        ],
        out_shape=[
            jax.ShapeDtypeStruct((_N, _D), jnp.float32),
            jax.ShapeDtypeStruct((8, _D), jnp.float32),
        ],
        scratch_shapes=[pltpu.VMEM((8, _D), jnp.float32)],
    )(eps.reshape(1), x, p0, p1, wa, ba.reshape(1, _D), wb, bb.reshape(1, _D))


def _bn_body(stats_ref, g_ref, b_ref, h_ref, out_ref):
    mu = stats_ref[0:1, :] * (1.0 / _N)
    var = stats_ref[1:2, :] * (1.0 / _N) - mu * mu
    scale = g_ref[...] * lax.rsqrt(var + _BN_EPS)
    shift = b_ref[...] - mu * scale
    out_ref[...] = h_ref[...] * scale + shift


@jax.jit
def _bn(stats, g, b, h):
    return pl.pallas_call(
        _bn_body,
        grid=(_NB,),
        in_specs=[
            pl.BlockSpec((8, _D), lambda i: (0, 0)),
            pl.BlockSpec((1, _D), lambda i: (0, 0)),
            pl.BlockSpec((1, _D), lambda i: (0, 0)),
            pl.BlockSpec((_BLK, _D), lambda i: (i, 0)),
        ],
        out_specs=pl.BlockSpec((_BLK, _D), lambda i: (i, 0)),
        out_shape=jax.ShapeDtypeStruct((_N, _D), jnp.float32),
    )(stats, g.reshape(1, _D), b.reshape(1, _D), h)


def _bn_lsm_body(stats_ref, g_ref, b_ref, h_ref, out_ref, lsm_ref):
    mu = stats_ref[0:1, :] * (1.0 / _N)
    var = stats_ref[1:2, :] * (1.0 / _N) - mu * mu
    scale = g_ref[...] * lax.rsqrt(var + _BN_EPS)
    shift = b_ref[...] - mu * scale
    h = h_ref[...] * scale + shift
    out_ref[...] = h
    m = jnp.max(h, axis=-1, keepdims=True)
    lse = jnp.log(jnp.sum(jnp.exp(h - m), axis=-1, keepdims=True)) + m
    lsm_ref[...] = h - lse


@jax.jit
def _bn_lsm(stats, g, b, h):
    return pl.pallas_call(
        _bn_lsm_body,
        grid=(_NB,),
        in_specs=[
            pl.BlockSpec((8, _D), lambda i: (0, 0)),
            pl.BlockSpec((1, _D), lambda i: (0, 0)),
            pl.BlockSpec((1, _D), lambda i: (0, 0)),
            pl.BlockSpec((_BLK, _D), lambda i: (i, 0)),
        ],
        out_specs=[
            pl.BlockSpec((_BLK, _D), lambda i: (i, 0)),
            pl.BlockSpec((_BLK, _D), lambda i: (i, 0)),
        ],
        out_shape=[
            jax.ShapeDtypeStruct((_N, _D), jnp.float32),
            jax.ShapeDtypeStruct((_N, _D), jnp.float32),
        ],
    )(stats, g.reshape(1, _D), b.reshape(1, _D), h)


def kernel(x, edge_index, W1_1, b1_1, W1_2, b1_2, bn1_g, bn1_b, eps1,
           W2_1, b2_1, W2_2, b2_2, bn2_g, bn2_b, eps2):
    edge_index = edge_index.astype(jnp.int32)
    pad = _EP - edge_index.shape[1]
    srcs = jnp.concatenate(
        [edge_index[0], jnp.zeros((pad,), jnp.int32)]
    ).reshape(_NW, _NCH // _HCH, _HCH, _CH)
    dsts = jnp.concatenate(
        [edge_index[1], jnp.full((pad,), _NP - 1, jnp.int32)]
    ).reshape(_NW, _NCH // _HCH, _HCH, _CH)
    zeros = jnp.zeros((_NP, _D), jnp.float32)

    p = _sc_agg(x, srcs, dsts, zeros)
    hpre1, stats1 = _mlp(eps1, x, p[0, :_N], p[1, :_N],
                         W1_1, b1_1, W1_2, b1_2)
    h1 = _bn(stats1, bn1_g, bn1_b, hpre1)

    p2 = _sc_agg(h1, srcs, dsts, zeros)
    hpre2, stats2 = _mlp(eps2, h1, p2[0, :_N], p2[1, :_N],
                         W2_1, b2_1, W2_2, b2_2)
    h2, lsm = _bn_lsm(stats2, bn2_g, bn2_b, hpre2)
    return (h2, lsm)


# speedup vs baseline: 2.5595x; 2.5595x over previous
import functools

import jax
import jax.numpy as jnp
from jax import lax
from jax.experimental import pallas as pl
from jax.experimental.pallas import tpu as pltpu
from jax.experimental.pallas import tpu_sc as plsc

_N = 10000
_D = 128
_NP = 10112
_NW = 32
_CH = 128
_NCH = 80
_HCH = 80
_EP = _NW * _NCH * _CH
_BN_EPS = 1e-5
_BLK = 1000
_NB = _N // _BLK


def _sc_agg_body(h_hbm, srcs_hbm, dsts_hbm, zeros_hbm, out_hbm,
                 src_v, dst_v, rows0_v, acc_sh, sem0):
    cid = lax.axis_index("c")
    sid = lax.axis_index("s")
    wid = sid * 2 + cid
    rz = _NP // 16

    pltpu.sync_copy(zeros_hbm.at[pl.ds(sid * rz, rz)],
                    acc_sh.at[pl.ds(sid * rz, rz)])
    plsc.subcore_barrier()

    def phase(p, carry):
        pltpu.sync_copy(srcs_hbm.at[wid, p], src_v)
        pltpu.sync_copy(dsts_hbm.at[wid, p], dst_v)

        def chunk(c, carry2):
            pltpu.async_copy(h_hbm.at[src_v.at[c]], rows0_v, sem0).wait()
            pltpu.sync_copy(rows0_v, acc_sh.at[dst_v.at[c]], add=True)
            return carry2

        lax.fori_loop(0, _HCH, chunk, 0)
        return carry

    lax.fori_loop(0, _NCH // _HCH, phase, 0)
    plsc.subcore_barrier()
    pltpu.sync_copy(acc_sh.at[pl.ds(sid * rz, rz)],
                    out_hbm.at[cid, pl.ds(sid * rz, rz)])


@jax.jit
def _sc_agg(h, srcs, dsts, zeros):
    mesh = plsc.VectorSubcoreMesh(core_axis_name="c", subcore_axis_name="s")
    return pl.kernel(
        _sc_agg_body,
        out_type=jax.ShapeDtypeStruct((2, _NP, _D), jnp.float32),
        mesh=mesh,
        scratch_types=[
            pltpu.VMEM((_HCH, _CH), jnp.int32),
            pltpu.VMEM((_HCH, _CH), jnp.int32),
            pltpu.VMEM((_CH, _D), jnp.float32),
            pltpu.VMEM_SHARED((_NP, _D), jnp.float32),
            pltpu.SemaphoreType.DMA,
        ],
    )(h, srcs, dsts, zeros)


def _mlp_body(eps_ref, x_ref, p0_ref, p1_ref, wa_ref, ba_ref, wb_ref, bb_ref,
              hpre_ref, stats_ref, acc_ref):
    h = x_ref[...] * (1.0 + eps_ref[0]) + p0_ref[...] + p1_ref[...]
    h = jnp.maximum(
        jnp.dot(h, wa_ref[...], preferred_element_type=jnp.float32)
        + ba_ref[...], 0.0)
    h = jnp.maximum(
        jnp.dot(h, wb_ref[...], preferred_element_type=jnp.float32)
        + bb_ref[...], 0.0)
    hpre_ref[...] = h
    i = pl.program_id(0)

    @pl.when(i == 0)
    def _():
        acc_ref[...] = jnp.zeros_like(acc_ref)

    acc_ref[0:1, :] += jnp.sum(h, axis=0, keepdims=True)
    acc_ref[1:2, :] += jnp.sum(h * h, axis=0, keepdims=True)

    @pl.when(i == pl.num_programs(0) - 1)
    def _():
        stats_ref[...] = acc_ref[...]


@jax.jit
def _mlp(eps, x, p0, p1, wa, ba, wb, bb):
    return pl.pallas_call(
        _mlp_body,
        grid=(_NB,),
        in_specs=[
            pl.BlockSpec(memory_space=pltpu.SMEM),
            pl.BlockSpec((_BLK, _D), lambda i: (i, 0)),
            pl.BlockSpec((_BLK, _D), lambda i: (i, 0)),
            pl.BlockSpec((_BLK, _D), lambda i: (i, 0)),
            pl.BlockSpec((_D, _D), lambda i: (0, 0)),
            pl.BlockSpec((1, _D), lambda i: (0, 0)),
            pl.BlockSpec((_D, _D), lambda i: (0, 0)),
            pl.BlockSpec((1, _D), lambda i: (0, 0)),
        ],
        out_specs=[
            pl.BlockSpec((_BLK, _D), lambda i: (i, 0)),
            pl.BlockSpec((8, _D), lambda i: (0, 0)),
        ],
        out_shape=[
            jax.ShapeDtypeStruct((_N, _D), jnp.float32),
            jax.ShapeDtypeStruct((8, _D), jnp.float32),
        ],
        scratch_shapes=[pltpu.VMEM((8, _D), jnp.float32)],
    )(eps.reshape(1), x, p0, p1, wa, ba.reshape(1, _D), wb, bb.reshape(1, _D))


def _bn_body(stats_ref, g_ref, b_ref, h_ref, out_ref):
    mu = stats_ref[0:1, :] * (1.0 / _N)
    var = stats_ref[1:2, :] * (1.0 / _N) - mu * mu
    scale = g_ref[...] * lax.rsqrt(var + _BN_EPS)
    shift = b_ref[...] - mu * scale
    out_ref[...] = h_ref[...] * scale + shift


@jax.jit
def _bn(stats, g, b, h):
    return pl.pallas_call(
        _bn_body,
        grid=(_NB,),
        in_specs=[
            pl.BlockSpec((8, _D), lambda i: (0, 0)),
            pl.BlockSpec((1, _D), lambda i: (0, 0)),
            pl.BlockSpec((1, _D), lambda i: (0, 0)),
            pl.BlockSpec((_BLK, _D), lambda i: (i, 0)),
        ],
        out_specs=pl.BlockSpec((_BLK, _D), lambda i: (i, 0)),
        out_shape=jax.ShapeDtypeStruct((_N, _D), jnp.float32),
    )(stats, g.reshape(1, _D), b.reshape(1, _D), h)


def _bn_lsm_body(stats_ref, g_ref, b_ref, h_ref, out_ref, lsm_ref):
    mu = stats_ref[0:1, :] * (1.0 / _N)
    var = stats_ref[1:2, :] * (1.0 / _N) - mu * mu
    scale = g_ref[...] * lax.rsqrt(var + _BN_EPS)
    shift = b_ref[...] - mu * scale
    h = h_ref[...] * scale + shift
    out_ref[...] = h
    m = jnp.max(h, axis=-1, keepdims=True)
    lse = jnp.log(jnp.sum(jnp.exp(h - m), axis=-1, keepdims=True)) + m
    lsm_ref[...] = h - lse


@jax.jit
def _bn_lsm(stats, g, b, h):
    return pl.pallas_call(
        _bn_lsm_body,
        grid=(_NB,),
        in_specs=[
            pl.BlockSpec((8, _D), lambda i: (0, 0)),
            pl.BlockSpec((1, _D), lambda i: (0, 0)),
            pl.BlockSpec((1, _D), lambda i: (0, 0)),
            pl.BlockSpec((_BLK, _D), lambda i: (i, 0)),
        ],
        out_specs=[
            pl.BlockSpec((_BLK, _D), lambda i: (i, 0)),
            pl.BlockSpec((_BLK, _D), lambda i: (i, 0)),
        ],
        out_shape=[
            jax.ShapeDtypeStruct((_N, _D), jnp.float32),
            jax.ShapeDtypeStruct((_N, _D), jnp.float32),
        ],
    )(stats, g.reshape(1, _D), b.reshape(1, _D), h)


def kernel(x, edge_index, W1_1, b1_1, W1_2, b1_2, bn1_g, bn1_b, eps1,
           W2_1, b2_1, W2_2, b2_2, bn2_g, bn2_b, eps2):
    edge_index = edge_index.astype(jnp.int32)
    pad = _EP - edge_index.shape[1]
    ar = jnp.arange(pad, dtype=jnp.int32)
    srcs = jnp.concatenate(
        [edge_index[0], ar % _N]
    ).reshape(_NW, _NCH // _HCH, _HCH, _CH)
    dsts = jnp.concatenate(
        [edge_index[1], _N + ar % (_NP - _N)]
    ).reshape(_NW, _NCH // _HCH, _HCH, _CH)
    zeros = jnp.zeros((_NP, _D), jnp.float32)

    p = _sc_agg(x, srcs, dsts, zeros)
    hpre1, stats1 = _mlp(eps1, x, p[0, :_N], p[1, :_N],
                         W1_1, b1_1, W1_2, b1_2)
    h1 = _bn(stats1, bn1_g, bn1_b, hpre1)

    p2 = _sc_agg(h1, srcs, dsts, zeros)
    hpre2, stats2 = _mlp(eps2, h1, p2[0, :_N], p2[1, :_N],
                         W2_1, b2_1, W2_2, b2_2)
    h2, lsm = _bn_lsm(stats2, bn2_g, bn2_b, hpre2)
    return (h2, lsm)
